# baseline (speedup 1.0000x reference)
import functools

import jax
import jax.numpy as jnp
from jax import lax
from jax.experimental import pallas as pl
from jax.experimental.pallas import tpu as pltpu
from jax.experimental.pallas import tpu_sc as plsc

N_NODES = 10000
N_EDGES = 320000
D = 128
L = 16
NW = 32
NPW = 320
N_PAD = NW * NPW
TRASH = NPW
C = 16000
N_CHUNKS = N_EDGES // C
GROUPS_PER_CHUNK = C // L

_NEG_INF = float("-inf")

_mesh = plsc.VectorSubcoreMesh(
    core_axis_name="c", subcore_axis_name="s", num_cores=2, num_subcores=16
)


@functools.partial(
    pl.kernel,
    out_type=jax.ShapeDtypeStruct((N_PAD, D), jnp.float32),
    mesh=_mesh,
    compiler_params=pltpu.CompilerParams(needs_layout_passes=False),
    scratch_types=[
        pltpu.VMEM((N_NODES,), jnp.float32),
        pltpu.VMEM((NPW + 1, D), jnp.float32),
        pltpu.VMEM((C,), jnp.int32),
        pltpu.VMEM((C,), jnp.int32),
        pltpu.VMEM((C + 16 * L,), jnp.int32),
        pltpu.VMEM((2, 4 * L, D), jnp.float32),
        pltpu.VMEM((2, 4 * L), jnp.int32),
        pltpu.SemaphoreType.DMA,
        pltpu.SemaphoreType.DMA,
    ],
)
def _sc_scatter_max(f_hbm, src_hbm, dst_hbm, norm_hbm, s_hbm,
                    norm_v, acc_v, srcc_v, dstc_v, midx_v, rows_v,
                    idx_v, sem0, sem1):
    wid = lax.axis_index("s") * 2 + lax.axis_index("c")
    lo = wid * NPW
    sems = (sem0, sem1)

    pltpu.sync_copy(norm_hbm, norm_v)

    def init_body(r, carry):
        for v in range(D // L):
            acc_v[r, pl.ds(v * L, L)] = jnp.full((L,), _NEG_INF, jnp.float32)
        return carry

    lax.fori_loop(0, NPW + 1, init_body, 0)

    lanes = lax.iota(jnp.int32, L)

    def chunk_body(c, carry):
        base = c * C
        pltpu.sync_copy(src_hbm.at[pl.ds(base, C)], srcc_v)
        pltpu.sync_copy(dst_hbm.at[pl.ds(base, C)], dstc_v)

        def scan_body(i, off):
            dv = dstc_v[pl.ds(i * L, L)]
            m = (dv >= lo) & (dv < lo + NPW)
            ids = lanes + i * L
            plsc.store_compressed(midx_v.at[pl.ds(off, L)], ids, mask=m)
            return off + plsc.all_reduce_population_count(m)[0]

        k = lax.fori_loop(0, GROUPS_PER_CHUNK, scan_body, jnp.int32(0))
        n_groups = (k + (L - 1)) // L
        QB = 4
        n_batchpairs = (n_groups + 2 * QB - 1) // (2 * QB)

        def prefetch(b, buf):
            infos = []
            for q in range(QB):
                g = b * QB + q
                mlane = (g * L + lanes) < k
                idxv = jnp.where(mlane, midx_v[pl.ds(g * L, L)], 0)
                srcs = plsc.load_gather(srcc_v, [idxv])
                dsts = plsc.load_gather(dstc_v, [idxv])
                w = plsc.load_gather(norm_v, [srcs]) * plsc.load_gather(norm_v, [dsts])
                dloc = jnp.where(mlane, dsts - lo, TRASH)
                idx_v[buf, pl.ds(q * L, L)] = srcs
                infos += [w, dloc]
            pltpu.make_async_copy(f_hbm.at[idx_v.at[buf]],
                                  rows_v.at[buf], sems[buf]).start()
            return tuple(infos)

        def compute(buf, infos):
            pltpu.make_async_copy(f_hbm.at[idx_v.at[buf]],
                                  rows_v.at[buf], sems[buf]).wait()
            for q in range(QB):
                w, dloc = infos[2 * q], infos[2 * q + 1]
                for j in range(L):
                    dj = dloc[j]
                    wv = jnp.full((L,), w[j], jnp.float32)
                    for v in range(D // L):
                        sl = pl.ds(v * L, L)
                        acc_v[dj, sl] = jnp.maximum(
                            acc_v[dj, sl], rows_v[buf, q * L + j, sl] * wv)

        @pl.when(n_groups > 0)
        def _():
            wd0 = prefetch(0, 0)

            def pair_body(p, carry2):
                wd1 = prefetch(2 * p + 1, 1)
                compute(0, carry2)
                wd0n = prefetch(2 * p + 2, 0)
                compute(1, wd1)
                return wd0n

            lax.fori_loop(0, n_batchpairs, pair_body, wd0)
            pltpu.make_async_copy(f_hbm.at[idx_v.at[0]],
                                  rows_v.at[0], sem0).wait()

        return carry

    lax.fori_loop(0, N_CHUNKS, chunk_body, 0)

    def fix_body(r, carry):
        for v in range(D // L):
            sl = pl.ds(v * L, L)
            a = acc_v[r, sl]
            acc_v[r, sl] = jnp.where(a == _NEG_INF, 0.0, a)
        return carry

    lax.fori_loop(0, NPW, fix_body, 0)

    pltpu.sync_copy(acc_v.at[pl.ds(0, NPW)], s_hbm.at[pl.ds(lo, NPW)])


def _tc_linear_body(s_ref, w_ref, o_ref):
    o_ref[...] = jnp.maximum(
        lax.dot_general(s_ref[...], w_ref[...], (((1,), (1,)), ((), ())),
                        preferred_element_type=jnp.float32),
        0.0,
    )


def _tc_linear(s_full, W):
    return pl.pallas_call(
        _tc_linear_body,
        out_shape=jax.ShapeDtypeStruct((N_PAD, D), jnp.float32),
    )(s_full, W)


def kernel(f, edge_index, norm, W):
    src = edge_index[0]
    dst = edge_index[1]
    s_full = _sc_scatter_max(f, src, dst, norm.reshape(-1))
    out_full = _tc_linear(s_full, W)
    return (out_full[:N_NODES], s_full[:N_NODES])


# speedup vs baseline: 1.1173x; 1.1173x over previous
import functools

import jax
import jax.numpy as jnp
from jax import lax
from jax.experimental import pallas as pl
from jax.experimental.pallas import tpu as pltpu
from jax.experimental.pallas import tpu_sc as plsc

N_NODES = 10000
N_EDGES = 320000
D = 128
L = 16
NW = 32
NPW = 320
N_PAD = NW * NPW
TRASH = NPW
C = 16000
N_CHUNKS = N_EDGES // C
GROUPS_PER_CHUNK = C // L

_NEG_INF = float("-inf")

_mesh = plsc.VectorSubcoreMesh(
    core_axis_name="c", subcore_axis_name="s", num_cores=2, num_subcores=16
)


@functools.partial(
    pl.kernel,
    out_type=jax.ShapeDtypeStruct((N_PAD, D), jnp.float32),
    mesh=_mesh,
    compiler_params=pltpu.CompilerParams(needs_layout_passes=False),
    scratch_types=[
        pltpu.VMEM((N_NODES,), jnp.float32),
        pltpu.VMEM((NPW + 1, D), jnp.float32),
        pltpu.VMEM((C,), jnp.int32),
        pltpu.VMEM((C,), jnp.int32),
        pltpu.VMEM((C + 16 * L,), jnp.int32),
        pltpu.VMEM((4, L, D), jnp.float32),
        pltpu.VMEM((4, L), jnp.int32),
        pltpu.VMEM((4, L), jnp.float32),
        pltpu.VMEM((4, L), jnp.int32),
        pltpu.SemaphoreType.DMA,
        pltpu.SemaphoreType.DMA,
        pltpu.SemaphoreType.DMA,
        pltpu.SemaphoreType.DMA,
    ],
)
def _sc_scatter_max(f_hbm, src_hbm, dst_hbm, norm_hbm, s_hbm,
                    norm_v, acc_v, srcc_v, dstc_v, midx_v, rows_v,
                    idx_v, wbuf_v, dbuf_v, sem0, sem1, sem2, sem3):
    wid = lax.axis_index("s") * 2 + lax.axis_index("c")
    lo = wid * NPW
    sems = (sem0, sem1, sem2, sem3)

    pltpu.sync_copy(norm_hbm, norm_v)

    def init_body(r, carry):
        for v in range(D // L):
            acc_v[r, pl.ds(v * L, L)] = jnp.full((L,), _NEG_INF, jnp.float32)
        return carry

    lax.fori_loop(0, NPW + 1, init_body, 0)

    lanes = lax.iota(jnp.int32, L)

    def chunk_body(c, carry):
        base = c * C
        pltpu.sync_copy(src_hbm.at[pl.ds(base, C)], srcc_v)
        pltpu.sync_copy(dst_hbm.at[pl.ds(base, C)], dstc_v)

        def scan_body(i, off):
            dv = dstc_v[pl.ds(i * L, L)]
            m = (dv >= lo) & (dv < lo + NPW)
            ids = lanes + i * L
            plsc.store_compressed(midx_v.at[pl.ds(off, L)], ids, mask=m)
            return off + plsc.all_reduce_population_count(m)[0]

        k = lax.fori_loop(0, GROUPS_PER_CHUNK, scan_body, jnp.int32(0))
        n_groups = (k + (L - 1)) // L
        NB = 4
        n_quads = (n_groups + NB - 1) // NB

        def prefetch(g, b):
            mlane = (g * L + lanes) < k
            idxv = jnp.where(mlane, midx_v[pl.ds(g * L, L)], 0)
            srcs = plsc.load_gather(srcc_v, [idxv])
            dsts = plsc.load_gather(dstc_v, [idxv])
            w = plsc.load_gather(norm_v, [srcs]) * plsc.load_gather(norm_v, [dsts])
            dloc = jnp.where(mlane, dsts - lo, TRASH)
            idx_v[b, :] = srcs
            wbuf_v[b, :] = w
            dbuf_v[b, :] = dloc
            pltpu.make_async_copy(f_hbm.at[idx_v.at[b]],
                                  rows_v.at[b], sems[b]).start()

        def consume(b):
            pltpu.make_async_copy(f_hbm.at[idx_v.at[b]],
                                  rows_v.at[b], sems[b]).wait()
            w = wbuf_v[b, :]
            dloc = dbuf_v[b, :]
            for j in range(L):
                dj = dloc[j]
                wv = jnp.full((L,), w[j], jnp.float32)
                for v in range(D // L):
                    sl = pl.ds(v * L, L)
                    acc_v[dj, sl] = jnp.maximum(acc_v[dj, sl],
                                                rows_v[b, j, sl] * wv)

        @pl.when(n_groups > 0)
        def _():
            for b in range(NB):
                prefetch(b, b)

            def quad_body(t, carry2):
                for b in range(NB):
                    consume(b)
                    prefetch(t * NB + NB + b, b)
                return carry2

            lax.fori_loop(0, n_quads, quad_body, 0)
            for b in range(NB):
                pltpu.make_async_copy(f_hbm.at[idx_v.at[b]],
                                      rows_v.at[b], sems[b]).wait()

        return carry

    lax.fori_loop(0, N_CHUNKS, chunk_body, 0)

    def fix_body(r, carry):
        for v in range(D // L):
            sl = pl.ds(v * L, L)
            a = acc_v[r, sl]
            acc_v[r, sl] = jnp.where(a == _NEG_INF, 0.0, a)
        return carry

    lax.fori_loop(0, NPW, fix_body, 0)

    pltpu.sync_copy(acc_v.at[pl.ds(0, NPW)], s_hbm.at[pl.ds(lo, NPW)])


def _tc_linear_body(s_ref, w_ref, o_ref):
    o_ref[...] = jnp.maximum(
        lax.dot_general(s_ref[...], w_ref[...], (((1,), (1,)), ((), ())),
                        preferred_element_type=jnp.float32),
        0.0,
    )


def _tc_linear(s_full, W):
    return pl.pallas_call(
        _tc_linear_body,
        out_shape=jax.ShapeDtypeStruct((N_PAD, D), jnp.float32),
    )(s_full, W)


def kernel(f, edge_index, norm, W):
    src = edge_index[0]
    dst = edge_index[1]
    s_full = _sc_scatter_max(f, src, dst, norm.reshape(-1))
    out_full = _tc_linear(s_full, W)
    return (out_full[:N_NODES], s_full[:N_NODES])


# speedup vs baseline: 1.8048x; 1.6152x over previous
import functools

import jax
import jax.numpy as jnp
from jax import lax
from jax.experimental import pallas as pl
from jax.experimental.pallas import tpu as pltpu
from jax.experimental.pallas import tpu_sc as plsc

N_NODES = 10000
N_EDGES = 320000
D = 128
L = 16
NW = 32
NPW = 320
N_PAD = NW * NPW
TRASH = NPW
C = 16000
N_CHUNKS = N_EDGES // C
GROUPS_PER_CHUNK = C // L

_NEG_INF = float("-inf")

_mesh = plsc.VectorSubcoreMesh(
    core_axis_name="c", subcore_axis_name="s", num_cores=2, num_subcores=16
)


@functools.partial(
    pl.kernel,
    out_type=jax.ShapeDtypeStruct((N_PAD, D), jnp.float32),
    mesh=_mesh,
    compiler_params=pltpu.CompilerParams(needs_layout_passes=False),
    scratch_types=[
        pltpu.VMEM((NPW + 1, D), jnp.float32),
        pltpu.VMEM((C,), jnp.int32),
        pltpu.VMEM((C,), jnp.int32),
        pltpu.VMEM((C + 3 * L,), jnp.int32),
        pltpu.VMEM((2, L, D), jnp.float32),
        pltpu.VMEM((2, L), jnp.int32),
        pltpu.SemaphoreType.DMA,
        pltpu.SemaphoreType.DMA,
    ],
)
def _sc_scatter_max(fn_hbm, src_hbm, dst_hbm, s_hbm,
                    acc_v, srcc_v, dstc_v, midx_v, rows_v,
                    idx_v, sem0, sem1):
    wid = lax.axis_index("s") * 2 + lax.axis_index("c")
    lo = wid * NPW
    sems = (sem0, sem1)

    def init_body(r, carry):
        for v in range(D // L):
            acc_v[r, pl.ds(v * L, L)] = jnp.full((L,), _NEG_INF, jnp.float32)
        return carry

    lax.fori_loop(0, NPW + 1, init_body, 0)

    lanes = lax.iota(jnp.int32, L)

    def chunk_body(c, carry):
        base = c * C
        pltpu.sync_copy(src_hbm.at[pl.ds(base, C)], srcc_v)
        pltpu.sync_copy(dst_hbm.at[pl.ds(base, C)], dstc_v)

        def scan_body(i, offv):
            dv = dstc_v[pl.ds(i * L, L)]
            m = (dv >= lo) & (dv < lo + NPW)
            ids = lanes + i * L
            pos = offv + plsc.cumsum(jnp.where(m, 1, 0)) - 1
            plsc.store_scatter(midx_v, [pos], ids, mask=m)
            return offv + plsc.all_reduce_population_count(m)

        offv = lax.fori_loop(0, GROUPS_PER_CHUNK, scan_body,
                             jnp.zeros((L,), jnp.int32), unroll=4)
        k = offv[0]
        n_groups = (k + (L - 1)) // L
        n_pairs = (n_groups + 1) // 2

        def prefetch(g, buf):
            mlane = (g * L + lanes) < k
            idxv = jnp.where(mlane, midx_v[pl.ds(g * L, L)], 0)
            srcs = plsc.load_gather(srcc_v, [idxv])
            dsts = plsc.load_gather(dstc_v, [idxv])
            dloc = jnp.where(mlane, dsts - lo, TRASH)
            idx_v[buf, :] = srcs
            pltpu.make_async_copy(fn_hbm.at[idx_v.at[buf]],
                                  rows_v.at[buf], sems[buf]).start()
            return dloc

        def compute(buf, dloc):
            pltpu.make_async_copy(fn_hbm.at[idx_v.at[buf]],
                                  rows_v.at[buf], sems[buf]).wait()
            for j in range(L):
                dj = dloc[j]
                for v in range(D // L):
                    sl = pl.ds(v * L, L)
                    acc_v[dj, sl] = jnp.maximum(acc_v[dj, sl],
                                                rows_v[buf, j, sl])

        @pl.when(n_groups > 0)
        def _():
            d0 = prefetch(0, 0)

            def pair_body(p, carry2):
                d1 = prefetch(2 * p + 1, 1)
                compute(0, carry2)
                d0n = prefetch(2 * p + 2, 0)
                compute(1, d1)
                return d0n

            lax.fori_loop(0, n_pairs, pair_body, d0)
            pltpu.make_async_copy(fn_hbm.at[idx_v.at[0]],
                                  rows_v.at[0], sem0).wait()

        return carry

    lax.fori_loop(0, N_CHUNKS, chunk_body, 0)

    def fix_body(r, carry):
        for v in range(D // L):
            sl = pl.ds(v * L, L)
            a = acc_v[r, sl]
            acc_v[r, sl] = jnp.where(a == _NEG_INF, 0.0, a)
        return carry

    lax.fori_loop(0, NPW, fix_body, 0)

    pltpu.sync_copy(acc_v.at[pl.ds(0, NPW)], s_hbm.at[pl.ds(lo, NPW)])


def _tc_prescale_body(f_ref, n_ref, o_ref):
    o_ref[...] = f_ref[...] * n_ref[...]


def _tc_prescale(f, norm_col):
    return pl.pallas_call(
        _tc_prescale_body,
        out_shape=jax.ShapeDtypeStruct((N_NODES, D), jnp.float32),
    )(f, norm_col)


def _tc_linear_body(sraw_ref, n_ref, w_ref, s_ref, o_ref):
    s = sraw_ref[...] * n_ref[...]
    s_ref[...] = s
    o_ref[...] = jnp.maximum(
        lax.dot_general(s, w_ref[...], (((1,), (1,)), ((), ())),
                        preferred_element_type=jnp.float32),
        0.0,
    )


def _tc_linear(s_raw, norm_pad, W):
    return pl.pallas_call(
        _tc_linear_body,
        out_shape=(jax.ShapeDtypeStruct((N_PAD, D), jnp.float32),
                   jax.ShapeDtypeStruct((N_PAD, D), jnp.float32)),
    )(s_raw, norm_pad, W)


def kernel(f, edge_index, norm, W):
    src = edge_index[0]
    dst = edge_index[1]
    norm_col = norm.reshape(N_NODES, 1)
    fn = _tc_prescale(f, norm_col)
    s_raw = _sc_scatter_max(fn, src, dst)
    norm_pad = jnp.zeros((N_PAD, 1), jnp.float32).at[:N_NODES].set(norm_col)
    s_full, out_full = _tc_linear(s_raw, norm_pad, W)
    return (out_full[:N_NODES], s_full[:N_NODES])


# speedup vs baseline: 2.8743x; 1.5926x over previous
import functools

import jax
import jax.numpy as jnp
from jax import lax
from jax.experimental import pallas as pl
from jax.experimental.pallas import tpu as pltpu
from jax.experimental.pallas import tpu_sc as plsc

N_NODES = 10000
N_EDGES = 320000
D = 128
L = 16
NW = 32
NPW = 320
N_PAD = NW * NPW
TRASH = NPW
C = 16000
N_CHUNKS = N_EDGES // C
GROUPS_PER_CHUNK = C // L

_NEG_INF = float("-inf")

_mesh = plsc.VectorSubcoreMesh(
    core_axis_name="c", subcore_axis_name="s", num_cores=2, num_subcores=16
)


@functools.partial(
    pl.kernel,
    out_type=jax.ShapeDtypeStruct((N_PAD, D), jnp.bfloat16),
    mesh=_mesh,
    compiler_params=pltpu.CompilerParams(needs_layout_passes=False),
    scratch_types=[
        pltpu.VMEM_SHARED((N_NODES, D // 2), jnp.int32),
        pltpu.VMEM((NPW + 1, D), jnp.bfloat16),
        pltpu.VMEM((C,), jnp.int32),
        pltpu.VMEM((C,), jnp.int32),
        pltpu.VMEM((C + 3 * L,), jnp.int32),
        pltpu.VMEM((2, L, D // 2), jnp.int32),
        pltpu.VMEM((2, L), jnp.int32),
        pltpu.SemaphoreType.DMA,
        pltpu.SemaphoreType.DMA,
    ],
)
def _sc_scatter_max(fn_hbm, src_hbm, dst_hbm, s_hbm,
                    shared_v, acc_v, srcc_v, dstc_v, midx_v, rows_v,
                    idx_v, sem0, sem1):
    wid = lax.axis_index("s") * 2 + lax.axis_index("c")
    lo = wid * NPW
    sems = (sem0, sem1)

    @pl.when(lax.axis_index("s") == 0)
    def _():
        pltpu.sync_copy(fn_hbm, shared_v)

    def init_body(r, carry):
        for v in range(D // 32):
            acc_v[r, pl.ds(v * 32, 32)] = jnp.full((32,), _NEG_INF,
                                                   jnp.bfloat16)
        return carry

    lax.fori_loop(0, NPW + 1, init_body, 0)
    plsc.subcore_barrier()

    lanes = lax.iota(jnp.int32, L)

    def chunk_body(c, carry):
        base = c * C
        pltpu.sync_copy(src_hbm.at[pl.ds(base, C)], srcc_v)
        pltpu.sync_copy(dst_hbm.at[pl.ds(base, C)], dstc_v)

        def scan_body(i, offv):
            dv = dstc_v[pl.ds(i * L, L)]
            m = (dv >= lo) & (dv < lo + NPW)
            ids = lanes + i * L
            pos = offv + plsc.cumsum(jnp.where(m, 1, 0)) - 1
            plsc.store_scatter(midx_v, [pos], ids, mask=m)
            return offv + plsc.all_reduce_population_count(m)

        offv = lax.fori_loop(0, GROUPS_PER_CHUNK, scan_body,
                             jnp.zeros((L,), jnp.int32), unroll=4)
        k = offv[0]
        n_groups = (k + (L - 1)) // L
        n_pairs = (n_groups + 1) // 2

        def prefetch(g, buf):
            mlane = (g * L + lanes) < k
            idxv = jnp.where(mlane, midx_v[pl.ds(g * L, L)], 0)
            srcs = plsc.load_gather(srcc_v, [idxv])
            dsts = plsc.load_gather(dstc_v, [idxv])
            dloc = jnp.where(mlane, dsts - lo, TRASH)
            idx_v[buf, :] = srcs
            pltpu.make_async_copy(shared_v.at[idx_v.at[buf]],
                                  rows_v.at[buf], sems[buf]).start()
            return dloc

        def compute(buf, dloc):
            pltpu.make_async_copy(shared_v.at[idx_v.at[buf]],
                                  rows_v.at[buf], sems[buf]).wait()
            for j in range(L):
                dj = dloc[j]
                for v in range(D // 32):
                    sl = pl.ds(v * 32, 32)
                    row = plsc.bitcast(rows_v[buf, j, pl.ds(v * L, L)],
                                       jnp.bfloat16)
                    acc_v[dj, sl] = jnp.maximum(acc_v[dj, sl], row)

        @pl.when(n_groups > 0)
        def _():
            d0 = prefetch(0, 0)

            def pair_body(p, carry2):
                d1 = prefetch(2 * p + 1, 1)
                compute(0, carry2)
                d0n = prefetch(2 * p + 2, 0)
                compute(1, d1)
                return d0n

            lax.fori_loop(0, n_pairs, pair_body, d0)
            pltpu.make_async_copy(shared_v.at[idx_v.at[0]],
                                  rows_v.at[0], sem0).wait()

        return carry

    lax.fori_loop(0, N_CHUNKS, chunk_body, 0)

    def fix_body(r, carry):
        for v in range(D // 32):
            sl = pl.ds(v * 32, 32)
            a = acc_v[r, sl]
            acc_v[r, sl] = jnp.where(a == jnp.bfloat16(_NEG_INF),
                                     jnp.bfloat16(0.0), a)
        return carry

    lax.fori_loop(0, NPW, fix_body, 0)

    pltpu.sync_copy(acc_v.at[pl.ds(0, NPW)], s_hbm.at[pl.ds(lo, NPW)])


def _tc_prescale_body(f_ref, n_ref, o_ref):
    o_ref[...] = (f_ref[...] * n_ref[...]).astype(jnp.bfloat16)


def _tc_prescale(f, norm_col):
    return pl.pallas_call(
        _tc_prescale_body,
        out_shape=jax.ShapeDtypeStruct((N_NODES, D), jnp.bfloat16),
    )(f, norm_col)


def _tc_linear_body(sraw_ref, n_ref, w_ref, s_ref, o_ref):
    s = sraw_ref[...].astype(jnp.float32) * n_ref[...]
    s_ref[...] = s
    o_ref[...] = jnp.maximum(
        lax.dot_general(s, w_ref[...], (((1,), (1,)), ((), ())),
                        preferred_element_type=jnp.float32),
        0.0,
    )


def _tc_linear(s_raw, norm_pad, W):
    return pl.pallas_call(
        _tc_linear_body,
        out_shape=(jax.ShapeDtypeStruct((N_PAD, D), jnp.float32),
                   jax.ShapeDtypeStruct((N_PAD, D), jnp.float32)),
    )(s_raw, norm_pad, W)


def kernel(f, edge_index, norm, W):
    src = edge_index[0]
    dst = edge_index[1]
    norm_col = norm.reshape(N_NODES, 1)
    fn = _tc_prescale(f, norm_col)
    fn_i32 = lax.bitcast_convert_type(fn.reshape(N_NODES, D // 2, 2),
                                      jnp.int32)
    s_raw = _sc_scatter_max(fn_i32, src, dst)
    norm_pad = jnp.zeros((N_PAD, 1), jnp.float32).at[:N_NODES].set(norm_col)
    s_full, out_full = _tc_linear(s_raw, norm_pad, W)
    return (out_full[:N_NODES], s_full[:N_NODES])
